# baseline (speedup 1.0000x reference)
import functools

import jax
import jax.numpy as jnp
from jax.experimental import pallas as pl
from jax.experimental.pallas import tpu as pltpu

ALPHA = 0.1
N = 16384
D = 64
BM = 256
NREF = 2
BMR = BM // NREF


def _prop_kernel(a0, a1, inp_ref, h_ref, out_ref):
    for r, a in enumerate((a0, a1)):
        sl = pl.ds(r * BMR, BMR)
        out_ref[sl, :] = (1.0 - ALPHA) * jnp.dot(
            a[...], inp_ref[...], preferred_element_type=jnp.float32
        ) + ALPHA * h_ref[sl, :]


@functools.partial(jax.jit, static_argnames=())
def kernel(input, adj, h, W):
    del W
    adj_specs = [
        pl.BlockSpec((BMR, N), lambda i, r=r: (NREF * i + r, 0))
        for r in range(NREF)
    ]
    return pl.pallas_call(
        _prop_kernel,
        grid=(N // BM,),
        in_specs=adj_specs + [
            pl.BlockSpec((N, D), lambda i: (0, 0)),
            pl.BlockSpec((BM, D), lambda i: (i, 0)),
        ],
        out_specs=pl.BlockSpec((BM, D), lambda i: (i, 0)),
        out_shape=jax.ShapeDtypeStruct((N, D), jnp.float32),
        compiler_params=pltpu.CompilerParams(
            dimension_semantics=("arbitrary",),
        ),
    )(adj, adj, input, h)


# speedup vs baseline: 1.0126x; 1.0126x over previous
import functools

import jax
import jax.numpy as jnp
from jax.experimental import pallas as pl
from jax.experimental.pallas import tpu as pltpu

ALPHA = 0.1
N = 16384
D = 64
BC = 64
NBUF = 4
NCH = N // BC


def _prop_kernel(adj_hbm, inp_ref, h_ref, out_ref, b0, b1, b2, b3, sems):
    bufs = (b0, b1, b2, b3)

    def chunk_copy(c, s):
        return pltpu.make_async_copy(
            adj_hbm.at[pl.ds(c * BC, BC), :], bufs[s], sems.at[s]
        )

    for s in range(NBUF):
        chunk_copy(s, s).start()

    def body(t, carry):
        for s in range(NBUF):
            c = t * NBUF + s
            chunk_copy(c, s).wait()
            rows = pl.ds(c * BC, BC)
            out_ref[rows, :] = (1.0 - ALPHA) * jnp.dot(
                bufs[s][...], inp_ref[...], preferred_element_type=jnp.float32
            ) + ALPHA * h_ref[rows, :]

            @pl.when(c + NBUF < NCH)
            def _refill():
                chunk_copy(c + NBUF, s).start()

        return carry

    jax.lax.fori_loop(0, NCH // NBUF, body, 0)


@functools.partial(jax.jit, static_argnames=())
def kernel(input, adj, h, W):
    del W
    return pl.pallas_call(
        _prop_kernel,
        in_specs=[
            pl.BlockSpec(memory_space=pltpu.MemorySpace.HBM),
            pl.BlockSpec(memory_space=pltpu.MemorySpace.VMEM),
            pl.BlockSpec(memory_space=pltpu.MemorySpace.VMEM),
        ],
        out_specs=pl.BlockSpec(memory_space=pltpu.MemorySpace.VMEM),
        out_shape=jax.ShapeDtypeStruct((N, D), jnp.float32),
        scratch_shapes=[
            pltpu.VMEM((BC, N), jnp.float32),
            pltpu.VMEM((BC, N), jnp.float32),
            pltpu.VMEM((BC, N), jnp.float32),
            pltpu.VMEM((BC, N), jnp.float32),
            pltpu.SemaphoreType.DMA((NBUF,)),
        ],
    )(adj, input, h)
